# baseline (speedup 1.0000x reference)
import functools

import jax
import jax.numpy as jnp
from jax import lax
from jax.experimental import pallas as pl
from jax.experimental.pallas import tpu as pltpu
from jax.experimental.pallas import tpu_sc as plsc

NC = 2
NS = 16
NW = NC * NS
LANES = 16
CHUNK = 128


@functools.lru_cache(maxsize=None)
def _build(n_tok: int, seq: int, d: int):
    bpw = n_tok // NW
    nch = bpw // CHUNK
    mesh = plsc.VectorSubcoreMesh(core_axis_name="c", subcore_axis_name="s")

    @functools.partial(
        pl.kernel,
        mesh=mesh,
        out_type=jax.ShapeDtypeStruct((n_tok, d), jnp.float32),
        scratch_types=[
            pltpu.VMEM((nch, CHUNK), jnp.int32),
            pltpu.VMEM((bpw, d), jnp.float32),
            pltpu.VMEM((bpw, d), jnp.float32),
            pltpu.SemaphoreType.DMA,
        ],
        compiler_params=pltpu.CompilerParams(use_tc_tiling_on_sc=False),
    )
    def emb(idx_hbm, tok_hbm, pos_hbm, out_hbm, idx_v, rows_v, pos_v, sem):
        wid = lax.axis_index("s") * NC + lax.axis_index("c")
        base = wid * bpw
        pos_base = lax.rem(base, seq)

        pltpu.sync_copy(idx_hbm.at[wid], idx_v)
        copies = []
        for j in range(nch):
            copies.append(
                pltpu.async_copy(
                    tok_hbm.at[idx_v.at[j]],
                    rows_v.at[pl.ds(j * CHUNK, CHUNK)],
                    sem,
                )
            )
        pltpu.sync_copy(pos_hbm.at[pl.ds(pos_base, bpw)], pos_v)
        for cp in copies:
            cp.wait()

        def row_body(i, carry):
            for c in range(d // LANES):
                sl = pl.ds(c * LANES, LANES)
                rows_v[i, sl] = rows_v[i, sl] + pos_v[i, sl]
            return carry

        lax.fori_loop(0, bpw, row_body, 0)

        pltpu.sync_copy(rows_v, out_hbm.at[pl.ds(base, bpw)])

    return emb


def kernel(inputs, token_table, pos_table):
    b, s = inputs.shape
    d = token_table.shape[1]
    n_tok = b * s
    idx = inputs.reshape(NW, (n_tok // NW) // CHUNK, CHUNK).astype(jnp.int32)
    out = _build(n_tok, s, d)(idx, token_table, pos_table)
    return out.reshape(b, s, d)


# speedup vs baseline: 1.6513x; 1.6513x over previous
import functools

import jax
import jax.numpy as jnp
from jax import lax
from jax.experimental import pallas as pl
from jax.experimental.pallas import tpu as pltpu
from jax.experimental.pallas import tpu_sc as plsc

NC = 2
NS = 16
NW = NC * NS
LANES = 16
WAVE = 64


@functools.lru_cache(maxsize=None)
def _build(n_tok: int, seq: int, d: int):
    bpw = n_tok // NW
    nwave = bpw // WAVE
    mesh = plsc.VectorSubcoreMesh(core_axis_name="c", subcore_axis_name="s")

    @functools.partial(
        pl.kernel,
        mesh=mesh,
        out_type=jax.ShapeDtypeStruct((n_tok, d), jnp.float32),
        scratch_types=[
            pltpu.VMEM((bpw,), jnp.int32),
            pltpu.VMEM((bpw, d), jnp.float32),
            pltpu.VMEM((bpw * d,), jnp.float32),
            pltpu.SemaphoreType.DMA,
            pltpu.SemaphoreType.DMA,
        ],
    )
    def emb(idx_hbm, tok_hbm, pos_hbm, out_hbm, idx_v, rows_v, pos_v, sem, psem):
        wid = lax.axis_index("s") * NC + lax.axis_index("c")
        base = wid * bpw
        pos_base = lax.rem(base, seq)

        pltpu.sync_copy(idx_hbm.at[wid], idx_v)
        pos_cp = pltpu.async_copy(
            pos_hbm.at[pl.ds(pos_base * d, bpw * d)], pos_v, psem
        )

        def wave_body(w, carry):
            wbase = w * WAVE
            for g in range(WAVE // LANES):
                gbase = wbase + g * LANES
                vec = idx_v[pl.ds(gbase, LANES)]
                for j in range(LANES):
                    pltpu.async_copy(tok_hbm.at[vec[j]], rows_v.at[gbase + j], sem)

            def drain(i, c):
                pltpu.make_async_copy(tok_hbm.at[0], rows_v.at[0], sem).wait()
                return c

            lax.fori_loop(0, WAVE, drain, 0, unroll=8)
            return carry

        lax.fori_loop(0, nwave, wave_body, 0)
        pos_cp.wait()

        def row_body(i, carry):
            for c in range(d // LANES):
                sl = pl.ds(c * LANES, LANES)
                rows_v[i, sl] = rows_v[i, sl] + pos_v[pl.ds(i * d + c * LANES, LANES)]
            return carry

        lax.fori_loop(0, bpw, row_body, 0)

        pltpu.sync_copy(rows_v, out_hbm.at[pl.ds(base, bpw)])

    return emb


def kernel(inputs, token_table, pos_table):
    b, s = inputs.shape
    d = token_table.shape[1]
    n_tok = b * s
    idx = inputs.reshape(NW, n_tok // NW).astype(jnp.int32)
    out = _build(n_tok, s, d)(idx, token_table, pos_table.reshape(-1))
    return out.reshape(b, s, d)


# speedup vs baseline: 1.9602x; 1.1871x over previous
import functools

import jax
import jax.numpy as jnp
from jax import lax
from jax.experimental import pallas as pl
from jax.experimental.pallas import tpu as pltpu
from jax.experimental.pallas import tpu_sc as plsc

NC = 2
NS = 16
NW = NC * NS
LANES = 16
WIN = 128
WAVE = 4


@functools.lru_cache(maxsize=None)
def _build(n_tok: int, seq: int, d: int):
    bpw = n_tok // NW
    nwave = bpw // WAVE
    mesh = plsc.VectorSubcoreMesh(core_axis_name="c", subcore_axis_name="s")

    @functools.partial(
        pl.kernel,
        mesh=mesh,
        out_type=jax.ShapeDtypeStruct((n_tok * d,), jnp.float32),
        scratch_types=[
            pltpu.VMEM((bpw,), jnp.int32),
            pltpu.VMEM((WAVE, d, WIN), jnp.float32),
            pltpu.VMEM((bpw * d,), jnp.float32),
            pltpu.VMEM((bpw * d,), jnp.float32),
            pltpu.SemaphoreType.DMA,
            pltpu.SemaphoreType.DMA,
        ],
        compiler_params=pltpu.CompilerParams(needs_layout_passes=False),
    )
    def emb(idx_hbm, tok_hbm, pos_hbm, out_hbm, idx_v, blk_v, rows_v, pos_v,
            sem, psem):
        wid = lax.axis_index("s") * NC + lax.axis_index("c")
        base = wid * bpw
        s0 = lax.rem(base, seq)

        pltpu.sync_copy(idx_hbm.at[wid], idx_v)
        pos_cp = pltpu.async_copy(
            pos_hbm.at[pl.ds(s0 * d, bpw * d)], pos_v, psem
        )

        ci = lax.iota(jnp.int32, LANES)

        def group_body(g, carry):
            gbase = g * LANES
            gvec = idx_v[pl.ds(gbase, LANES)]
            vwin = jnp.left_shift(jnp.right_shift(gvec, 7), 7)
            vlane = jnp.bitwise_and(gvec, WIN - 1)
            for wv in range(LANES // WAVE):
                for j in range(WAVE):
                    win = pl.multiple_of(vwin[wv * WAVE + j], WIN)
                    pltpu.async_copy(
                        tok_hbm.at[:, pl.ds(win, WIN)], blk_v.at[j], sem
                    )
                for j in range(WAVE):
                    pltpu.make_async_copy(
                        tok_hbm.at[:, pl.ds(0, WIN)], blk_v.at[0], sem
                    ).wait()
                for j in range(WAVE):
                    t = gbase + wv * WAVE + j
                    ti = jnp.full((LANES,), j, jnp.int32)
                    li = jnp.full((LANES,), vlane[wv * WAVE + j], jnp.int32)
                    for c in range(d // LANES):
                        vals = plsc.load_gather(
                            blk_v, [ti, c * LANES + ci, li]
                        )
                        sl = pl.ds(t * d + c * LANES, LANES)
                        rows_v[sl] = vals + pos_v[sl]
            return carry

        pos_cp.wait()
        lax.fori_loop(0, bpw // LANES, group_body, 0)

        pltpu.sync_copy(rows_v, out_hbm.at[pl.ds(base * d, bpw * d)])

    return emb


def kernel(inputs, token_table, pos_table):
    b, s = inputs.shape
    d = token_table.shape[1]
    n_tok = b * s
    idx = inputs.reshape(NW, n_tok // NW).astype(jnp.int32)
    out = _build(n_tok, s, d)(idx, token_table.T, pos_table.reshape(-1))
    return out.reshape(b, s, d)


# speedup vs baseline: 2.2719x; 1.1590x over previous
import functools

import jax
import jax.numpy as jnp
from jax import lax
from jax.experimental import pallas as pl
from jax.experimental.pallas import tpu as pltpu
from jax.experimental.pallas import tpu_sc as plsc

NC = 2
NS = 16
NW = NC * NS
LANES = 16
WIN = 128
WAVE = 4


@functools.lru_cache(maxsize=None)
def _build(n_tok: int, seq: int, d: int):
    bpw = n_tok // NW
    mesh = plsc.VectorSubcoreMesh(core_axis_name="c", subcore_axis_name="s")

    @functools.partial(
        pl.kernel,
        mesh=mesh,
        out_type=jax.ShapeDtypeStruct((n_tok * d,), jnp.float32),
        scratch_types=[
            pltpu.VMEM((bpw,), jnp.int32),
            pltpu.VMEM((2 * WAVE, d, WIN), jnp.float32),
            pltpu.VMEM((bpw * d,), jnp.float32),
            pltpu.VMEM((LANES * d,), jnp.float32),
            pltpu.SemaphoreType.DMA,
            pltpu.SemaphoreType.DMA,
            pltpu.SemaphoreType.DMA,
        ],
        compiler_params=pltpu.CompilerParams(needs_layout_passes=False),
    )
    def emb(idx_hbm, tok_hbm, pos_hbm, out_hbm, idx_v, blk_v, rows_v, pos_v,
            semA, semB, psem):
        wid = lax.axis_index("s") * NC + lax.axis_index("c")
        base = wid * bpw
        s0 = lax.rem(base, seq)

        pltpu.sync_copy(idx_hbm.at[wid], idx_v)
        sems = (semA, semB)
        ci = lax.iota(jnp.int32, LANES)

        def fire(vwin, wv, buf):
            for j in range(WAVE):
                win = pl.multiple_of(vwin[wv * WAVE + j], WIN)
                pltpu.async_copy(
                    tok_hbm.at[:, pl.ds(win, WIN)],
                    blk_v.at[buf * WAVE + j],
                    sems[buf],
                )

        def drain(buf):
            for _ in range(WAVE):
                pltpu.make_async_copy(
                    tok_hbm.at[:, pl.ds(0, WIN)], blk_v.at[0], sems[buf]
                ).wait()

        def extract(gbase, vlane, wv, buf):
            for j in range(WAVE):
                t = wv * WAVE + j
                ti = jnp.full((LANES,), buf * WAVE + j, jnp.int32)
                li = jnp.full((LANES,), vlane[t], jnp.int32)
                for c in range(d // LANES):
                    vals = plsc.load_gather(blk_v, [ti, c * LANES + ci, li])
                    sl = pl.ds((gbase + t) * d + c * LANES, LANES)
                    rows_v[sl] = vals + pos_v[pl.ds(t * d + c * LANES, LANES)]

        def group_body(g, carry):
            gbase = g * LANES
            gvec = idx_v[pl.ds(gbase, LANES)]
            vwin = jnp.left_shift(jnp.right_shift(gvec, 7), 7)
            vlane = jnp.bitwise_and(gvec, WIN - 1)
            pos_cp = pltpu.async_copy(
                pos_hbm.at[pl.ds((s0 + gbase) * d, LANES * d)], pos_v, psem
            )
            fire(vwin, 0, 0)
            fire(vwin, 1, 1)
            pos_cp.wait()
            drain(0)
            extract(gbase, vlane, 0, 0)
            fire(vwin, 2, 0)
            drain(1)
            extract(gbase, vlane, 1, 1)
            fire(vwin, 3, 1)
            drain(0)
            extract(gbase, vlane, 2, 0)
            drain(1)
            extract(gbase, vlane, 3, 1)
            return carry

        lax.fori_loop(0, bpw // LANES, group_body, 0)

        pltpu.sync_copy(rows_v, out_hbm.at[pl.ds(base * d, bpw * d)])

    return emb


def kernel(inputs, token_table, pos_table):
    b, s = inputs.shape
    d = token_table.shape[1]
    n_tok = b * s
    idx = inputs.reshape(NW, n_tok // NW).astype(jnp.int32)
    out = _build(n_tok, s, d)(idx, token_table.T, pos_table.reshape(-1))
    return out.reshape(b, s, d)
